# Initial kernel scaffold; baseline (speedup 1.0000x reference)
import jax
import jax.numpy as jnp
from jax.experimental import pallas as pl


def kernel(tokens, num_array, table, ln_gamma, ln_beta):
    raise NotImplementedError("write your pallas kernel here")


# speedup vs baseline: 2.9128x; 2.9128x over previous
import dataclasses
import functools

import jax
import jax.numpy as jnp
from jax.experimental import pallas as pl
from jax.experimental.pallas import tpu as pltpu
from jax.experimental.pallas import tpu_sc as plsc

EMB = 64
_SCALE = 8.0
_EPS = 1e-5
_LANES = 16
_W = 128
_ROWS = 2000


def _ln_body(tab_ref, g_ref, b_ref, o_ref):
    x = tab_ref[...] * _SCALE
    mean = jnp.mean(x, axis=-1, keepdims=True)
    xc = x - mean
    var = jnp.mean(xc * xc, axis=-1, keepdims=True)
    xhat = xc / jnp.sqrt(var + _EPS)
    o_ref[...] = xhat * g_ref[...] + b_ref[...]


def _normalize_table(table, gamma, beta):
    vocab = table.shape[0]
    grid = vocab // _ROWS
    return pl.pallas_call(
        _ln_body,
        grid=(grid,),
        in_specs=[
            pl.BlockSpec((_ROWS, EMB), lambda i: (i, 0)),
            pl.BlockSpec((1, EMB), lambda i: (0, 0)),
            pl.BlockSpec((1, EMB), lambda i: (0, 0)),
        ],
        out_specs=pl.BlockSpec((_ROWS, EMB), lambda i: (i, 0)),
        out_shape=jax.ShapeDtypeStruct((vocab, EMB), jnp.float32),
    )(table, gamma.reshape(1, EMB), beta.reshape(1, EMB))


def _sc_gather_scale(ntab, tok2d, num2d, n):
    grid = n // _W
    mesh = plsc.VectorSubcoreMesh(core_axis_name="c", subcore_axis_name="s")
    cp = pltpu.CompilerParams(
        needs_layout_passes=False, use_tc_tiling_on_sc=False
    )

    @functools.partial(
        pl.kernel,
        out_type=jax.ShapeDtypeStruct((n, EMB), jnp.float32),
        mesh=mesh,
        compiler_params=cp,
    )
    def run(tab_hbm, tok_hbm, num_hbm, out_hbm):
        def body(tok_v, num_v, o_v):
            pltpu.sync_copy(tab_hbm.at[tok_v.at[0]], o_v)

            @pl.loop(0, _W)
            def _(r):
                s = plsc.load_gather(
                    num_v,
                    [jnp.zeros((_LANES,), jnp.int32),
                     jnp.full((_LANES,), r, jnp.int32)],
                )
                for c in range(EMB // _LANES):
                    sl = (r, pl.ds(c * _LANES, _LANES))
                    o_v[sl] = o_v[sl] * s

        pltpu.emit_pipeline(
            body,
            grid=(grid,),
            in_specs=[
                pl.BlockSpec((1, _W), lambda i: (0, i)),
                pl.BlockSpec((1, _W), lambda i: (0, i)),
            ],
            out_specs=[pl.BlockSpec((_W, EMB), lambda i: (i, 0))],
            core_axis_name=("c", "s"),
            dimension_semantics=(pltpu.PARALLEL,),
        )(tok_hbm, num_hbm, out_hbm)

    return run(ntab, tok2d, num2d)


def kernel(tokens, num_array, table, ln_gamma, ln_beta):
    b, l = tokens.shape
    n = b * l
    ntab = _normalize_table(table, ln_gamma, ln_beta)
    tok2d = tokens.reshape(1, n).astype(jnp.int32)
    num2d = num_array.reshape(1, n)
    out = _sc_gather_scale(ntab, tok2d, num2d, n)
    return out.reshape(b, l, EMB)
